# Initial kernel scaffold; baseline (speedup 1.0000x reference)
import functools

import jax
import jax.numpy as jnp
from jax import lax
from jax.experimental import pallas as pl
from jax.experimental.pallas import tpu as pltpu
from jax.experimental.pallas import tpu_sc as plsc

B = 16384
N_SPARSE = 26
N_DENSE = 7
N_COLS = N_SPARSE + N_DENSE
VOCAB = 1000

NUM_CORES = 2
NUM_SUBCORES = 16
NW = NUM_CORES * NUM_SUBCORES
ROWS_PER_W = B // NW
XW_WORDS = ROWS_PER_W * N_COLS
TABLE_WORDS = N_SPARSE * VOCAB
LANES = 16
GROUPS = ROWS_PER_W // LANES


@functools.partial(
    pl.kernel,
    mesh=plsc.VectorSubcoreMesh(core_axis_name="c", subcore_axis_name="s"),
    out_type=jax.ShapeDtypeStruct((B,), jnp.float32),
    scratch_types=[
        pltpu.VMEM((XW_WORDS,), jnp.float32),
        pltpu.VMEM((TABLE_WORDS,), jnp.float32),
        pltpu.VMEM((8,), jnp.float32),
        pltpu.VMEM((ROWS_PER_W,), jnp.float32),
    ],
)
def _linear_logit_sc(x_hbm, t_hbm, w_hbm, out_hbm, xv, tv, wv, ov):
    wid = lax.axis_index("s") * NUM_CORES + lax.axis_index("c")
    base = wid * ROWS_PER_W
    pltpu.sync_copy(x_hbm.at[pl.ds(base * N_COLS, XW_WORDS)], xv)
    pltpu.sync_copy(t_hbm, tv)
    pltpu.sync_copy(w_hbm, wv)

    wsplat = [
        plsc.load_gather(wv, [jnp.full((LANES,), d, jnp.int32)])
        for d in range(N_DENSE)
    ]
    lanes = lax.iota(jnp.int32, (LANES,), 0)

    def group(g, carry):
        rows = g * LANES + lanes
        xbase = rows * N_COLS
        acc = jnp.zeros((LANES,), jnp.float32)
        for f in range(N_SPARSE):
            idf = plsc.load_gather(xv, [xbase + f])
            ids = idf.astype(jnp.int32) + f * VOCAB
            acc = acc + plsc.load_gather(tv, [ids])
        for d in range(N_DENSE):
            xd = plsc.load_gather(xv, [xbase + (N_SPARSE + d)])
            acc = acc + xd * wsplat[d]
        ov[pl.ds(g * LANES, LANES)] = acc
        return carry

    lax.fori_loop(0, GROUPS, group, 0)
    pltpu.sync_copy(ov, out_hbm.at[pl.ds(base, ROWS_PER_W)])


def kernel(X, emb_tables, dense_weight):
    x_flat = X.reshape(-1)
    t_flat = emb_tables.reshape(-1)
    w_pad = jnp.pad(dense_weight.reshape(-1), (0, 8 - N_DENSE))
    out = _linear_logit_sc(x_flat, t_flat, w_pad)
    return out.reshape(B, 1)


# speedup vs baseline: 64.4427x; 64.4427x over previous
import functools

import jax
import jax.numpy as jnp
from jax import lax
from jax.experimental import pallas as pl
from jax.experimental.pallas import tpu as pltpu
from jax.experimental.pallas import tpu_sc as plsc

B = 16384
N_SPARSE = 26
N_DENSE = 7
N_COLS = N_SPARSE + N_DENSE
VOCAB = 1000

NUM_CORES = 2
NUM_SUBCORES = 16
NW = NUM_CORES * NUM_SUBCORES
ROWS_PER_W = B // NW
XW_WORDS = ROWS_PER_W * N_COLS
TABLE_WORDS = N_SPARSE * VOCAB
TBUF_WORDS = TABLE_WORDS + 8
LANES = 16
GROUPS = ROWS_PER_W // LANES


@functools.partial(
    pl.kernel,
    mesh=plsc.VectorSubcoreMesh(core_axis_name="c", subcore_axis_name="s"),
    out_type=jax.ShapeDtypeStruct((B,), jnp.float32),
    compiler_params=pltpu.CompilerParams(needs_layout_passes=False),
    scratch_types=[
        pltpu.VMEM((XW_WORDS,), jnp.float32),
        pltpu.VMEM((TBUF_WORDS,), jnp.float32),
        pltpu.VMEM((ROWS_PER_W,), jnp.float32),
    ],
)
def _linear_logit_sc(x_hbm, t_hbm, out_hbm, xv, tv, ov):
    wid = lax.axis_index("s") * NUM_CORES + lax.axis_index("c")
    base = wid * ROWS_PER_W
    pltpu.sync_copy(x_hbm.at[pl.ds(base * N_COLS, XW_WORDS)], xv)
    pltpu.sync_copy(t_hbm, tv)

    wsplat = [
        plsc.load_gather(tv, [jnp.full((LANES,), TABLE_WORDS + d, jnp.int32)])
        for d in range(N_DENSE)
    ]
    lanes = lax.broadcasted_iota(jnp.int32, (LANES,), 0)

    def group(g, carry):
        rows = g * LANES + lanes
        xbase = rows * N_COLS
        acc = jnp.zeros((LANES,), jnp.float32)
        for f in range(N_SPARSE):
            idf = plsc.load_gather(xv, [xbase + f])
            ids = idf.astype(jnp.int32) + f * VOCAB
            acc = acc + plsc.load_gather(tv, [ids])
        for d in range(N_DENSE):
            xd = plsc.load_gather(xv, [xbase + (N_SPARSE + d)])
            acc = acc + xd * wsplat[d]
        ov[pl.ds(g * LANES, LANES)] = acc
        return carry

    lax.fori_loop(0, GROUPS, group, 0)
    pltpu.sync_copy(ov, out_hbm.at[pl.ds(base, ROWS_PER_W)])


def kernel(X, emb_tables, dense_weight):
    x_flat = X.reshape(-1)
    t_flat = jnp.concatenate([
        emb_tables.reshape(-1),
        jnp.pad(dense_weight.reshape(-1), (0, 8 - N_DENSE)),
    ])
    out = _linear_logit_sc(x_flat, t_flat)
    return out.reshape(B, 1)
